# baseline (speedup 1.0000x reference)
import functools

import jax
import jax.numpy as jnp
from jax.experimental import pallas as pl
from jax.experimental.pallas import tpu as pltpu

N = 50000
VOCAB = 512
HID = 64
H1, C1 = 8, 8
H2, C2 = 4, 16
NG = 64
NC = 20

ROW_BLK = 400


def _proj_body(x_ref, w_ref, b_ref, o_ref):
    o_ref[...] = jnp.dot(x_ref[...], w_ref[...],
                         preferred_element_type=jnp.float32) + b_ref[...]


def _proj(x, Wp, bp):
    return pl.pallas_call(
        _proj_body,
        grid=(N // ROW_BLK,),
        in_specs=[
            pl.BlockSpec((ROW_BLK, VOCAB), lambda i: (i, 0)),
            pl.BlockSpec((VOCAB, HID), lambda i: (0, 0)),
            pl.BlockSpec((1, HID), lambda i: (0, 0)),
        ],
        out_specs=pl.BlockSpec((ROW_BLK, HID), lambda i: (i, 0)),
        out_shape=jax.ShapeDtypeStruct((N, HID), jnp.float32),
    )(x, Wp, bp.reshape(1, HID))


def _layer_norm(x, g, b, eps=1e-5):
    m = x.mean(-1, keepdims=True)
    v = ((x - m) ** 2).mean(-1, keepdims=True)
    return (x - m) / jnp.sqrt(v + eps) * g + b


def _gat_conv(x, src, dst, W, a_src_p, a_dst_p, bias, heads, out_ch):
    n = x.shape[0]
    h = (x @ W).reshape(n, heads, out_ch)
    a_src = (h * a_src_p).sum(-1)
    a_dst = (h * a_dst_p).sum(-1)
    e = jax.nn.leaky_relu(a_src[src] + a_dst[dst], 0.2)
    e_max = jax.ops.segment_max(e, dst, num_segments=n)
    e_max = jnp.where(jnp.isfinite(e_max), e_max, 0.0)
    ex = jnp.exp(e - e_max[dst])
    denom = jax.ops.segment_sum(ex, dst, num_segments=n)
    alpha = ex / (denom[dst] + 1e-16)
    out = jax.ops.segment_sum(h[src] * alpha[..., None], dst, num_segments=n)
    return out.reshape(n, heads * out_ch) + bias


def kernel(x, edge_index, batch, Wp, bp, W1, as1, ad1, b1, g1, be1, W2, as2,
           ad2, b2, g2, be2, Wf, bf, Wc1, bc1, Wc2, bc2):
    n = x.shape[0]
    loops = jnp.arange(n)
    src = jnp.concatenate([edge_index[0], loops])
    dst = jnp.concatenate([edge_index[1], loops])
    h = _proj(x, Wp, bp)
    input_features = h
    a1 = jax.nn.elu(_gat_conv(h, src, dst, W1, as1, ad1, b1, H1, C1))
    h = _layer_norm(a1 + h, g1, be1)
    a2 = jax.nn.elu(_gat_conv(h, src, dst, W2, as2, ad2, b2, H2, C2))
    h = _layer_norm(a2 + h, g2, be2)
    ones = jnp.ones((n, 1), h.dtype)
    cnt = jnp.maximum(jax.ops.segment_sum(ones, batch, num_segments=NG), 1.0)
    xg = jax.ops.segment_sum(h, batch, num_segments=NG) / cnt
    xs = jax.ops.segment_sum(input_features, batch, num_segments=NG) / cnt
    z = jnp.concatenate([xg, xs], axis=-1)
    z = jax.nn.relu(z @ Wf + bf)
    z = jax.nn.relu(z @ Wc1 + bc1)
    z = z @ Wc2 + bc2
    return jax.nn.log_softmax(z, axis=1)


# speedup vs baseline: 1.0000x; 1.0000x over previous
import jax
import jax.numpy as jnp
from jax.experimental import pallas as pl

N = 50000
VOCAB = 512
HID = 64
H1, C1 = 8, 8
H2, C2 = 4, 16
NG = 64
NC = 20

ROW_BLK = 400


def _proj_body(x_ref, w_ref, b_ref, o_ref):
    o_ref[...] = jnp.dot(x_ref[...], w_ref[...],
                         preferred_element_type=jnp.float32) + b_ref[...]


def _proj(x, Wp, bp):
    return pl.pallas_call(
        _proj_body,
        grid=(N // ROW_BLK,),
        in_specs=[
            pl.BlockSpec((ROW_BLK, VOCAB), lambda i: (i, 0)),
            pl.BlockSpec((VOCAB, HID), lambda i: (0, 0)),
            pl.BlockSpec((1, HID), lambda i: (0, 0)),
        ],
        out_specs=pl.BlockSpec((ROW_BLK, HID), lambda i: (i, 0)),
        out_shape=jax.ShapeDtypeStruct((N, HID), jnp.float32),
    )(x, Wp, bp.reshape(1, HID))


def _layer_norm(x, g, b, eps=1e-5):
    m = x.mean(-1, keepdims=True)
    v = ((x - m) ** 2).mean(-1, keepdims=True)
    return (x - m) / jnp.sqrt(v + eps) * g + b


def _gat_conv(x, src, dst, W, a_src_p, a_dst_p, bias, heads, out_ch):
    n = x.shape[0]
    h = (x @ W).reshape(n, heads, out_ch)
    a_src = (h * a_src_p).sum(-1)
    a_dst = (h * a_dst_p).sum(-1)
    e = jax.nn.leaky_relu(a_src[src] + a_dst[dst], 0.2)
    e_max = jax.ops.segment_max(e, dst, num_segments=n)
    e_max = jnp.where(jnp.isfinite(e_max), e_max, 0.0)
    ex = jnp.exp(e - e_max[dst])
    denom = jax.ops.segment_sum(ex, dst, num_segments=n)
    alpha = ex / (denom[dst] + 1e-16)
    out = jax.ops.segment_sum(h[src] * alpha[..., None], dst, num_segments=n)
    return out.reshape(n, heads * out_ch) + bias


def kernel(x, edge_index, batch, Wp, bp, W1, as1, ad1, b1, g1, be1, W2, as2,
           ad2, b2, g2, be2, Wf, bf, Wc1, bc1, Wc2, bc2):
    n = x.shape[0]
    loops = jnp.arange(n)
    src = jnp.concatenate([edge_index[0], loops])
    dst = jnp.concatenate([edge_index[1], loops])
    h = _proj(x, Wp, bp)
    input_features = h
    a1 = jax.nn.elu(_gat_conv(h, src, dst, W1, as1, ad1, b1, H1, C1))
    h = _layer_norm(a1 + h, g1, be1)
    a2 = jax.nn.elu(_gat_conv(h, src, dst, W2, as2, ad2, b2, H2, C2))
    h = _layer_norm(a2 + h, g2, be2)
    ones = jnp.ones((n, 1), h.dtype)
    cnt = jnp.maximum(jax.ops.segment_sum(ones, batch, num_segments=NG), 1.0)
    xg = jax.ops.segment_sum(h, batch, num_segments=NG) / cnt
    xs = jax.ops.segment_sum(input_features, batch, num_segments=NG) / cnt
    z = jnp.concatenate([xg, xs], axis=-1)
    z = jax.nn.relu(z @ Wf + bf)
    z = jax.nn.relu(z @ Wc1 + bc1)
    z = z @ Wc2 + bc2
    return jax.nn.log_softmax(z, axis=1)
